# baseline (speedup 1.0000x reference)
import functools

import jax
import jax.numpy as jnp
from jax import lax
from jax.experimental import pallas as pl
from jax.experimental.pallas import tpu as pltpu
from jax.experimental.pallas import tpu_sc as plsc

N_NODES = 10000
F = 128
FH = F // 2
N_EDGES = 320000
NC = 2
NS = 16
EPW = N_EDGES // NS
K = 80
NCH = EPW // K
N_PAD = 10240
STRIPE = N_PAD // NS


NB = 5
NG = NCH // NB


def _spmm_body(mat0_hbm, mat1_hbm, src_hbm, dst_hbm, vals_hbm, zero_hbm,
               out_hbm, src_v, dst_v, vals_v,
               r0, r1, r2, r3, r4, acc_sh,
               g0, g1, g2, g3, g4, s0, s1, s2, s3, s4):
    c = lax.axis_index("c")
    s = lax.axis_index("s")
    rows = [r0, r1, r2, r3, r4]
    gsem = [g0, g1, g2, g3, g4]
    ssem = [s0, s1, s2, s3, s4]

    pltpu.sync_copy(src_hbm.at[s], src_v)
    pltpu.sync_copy(dst_hbm.at[s], dst_v)
    pltpu.sync_copy(vals_hbm.at[s], vals_v)
    pltpu.sync_copy(zero_hbm.at[pl.ds(s * STRIPE, STRIPE)],
                    acc_sh.at[pl.ds(s * STRIPE, STRIPE)])
    plsc.subcore_barrier()

    def gather(j, b):
        @pl.when(c == 0)
        def _():
            pltpu.async_copy(mat0_hbm.at[src_v.at[j]], rows[b], gsem[b])

        @pl.when(c == 1)
        def _():
            pltpu.async_copy(mat1_hbm.at[src_v.at[j]], rows[b], gsem[b])

    def wait_gather(b):
        pltpu.make_async_copy(mat0_hbm.at[src_v.at[0]], rows[b],
                              gsem[b]).wait()

    def scatter(j, b):
        pltpu.async_copy(rows[b], acc_sh.at[dst_v.at[j]], ssem[b],
                         add=True)

    def wait_scatter(b):
        pltpu.make_async_copy(rows[b], acc_sh.at[dst_v.at[0]],
                              ssem[b]).wait()

    def scale(j, b):
        @plsc.parallel_loop(0, K // 16, unroll=K // 16)
        def grp_body(g):
            vv = vals_v[j, pl.ds(g * 16, 16)]
            for l in range(16):
                v = vv[l]
                e = g * 16 + l
                for blk in range(FH // 16):
                    sl = pl.ds(blk * 16, 16)
                    rows[b][e, sl] = rows[b][e, sl] * v

    def group_body(gi, carry):
        base = gi * NB

        @pl.when(gi > 0)
        def _():
            with jax.named_scope("ws"):
                for b in range(NB):
                    wait_scatter(b)

        with jax.named_scope("gi"):
            for b in range(NB):
                gather(base + b, b)
        for b in range(NB):
            with jax.named_scope("wg"):
                wait_gather(b)
            with jax.named_scope("sc"):
                scale(base + b, b)
            with jax.named_scope("st"):
                scatter(base + b, b)
        return carry

    lax.fori_loop(0, NG, group_body, 0)
    for b in range(NB):
        wait_scatter(b)
    plsc.subcore_barrier()

    pltpu.sync_copy(acc_sh.at[pl.ds(s * STRIPE, STRIPE)],
                    out_hbm.at[c, pl.ds(s * STRIPE, STRIPE)])


_sc_spmm = functools.partial(
    pl.kernel,
    out_type=jax.ShapeDtypeStruct((NC, N_PAD, FH), jnp.float32),
    mesh=plsc.VectorSubcoreMesh(core_axis_name="c", subcore_axis_name="s"),
    compiler_params=pltpu.CompilerParams(use_tc_tiling_on_sc=False),
    scratch_types=(
        [
            pltpu.VMEM((NCH, K), jnp.int32),
            pltpu.VMEM((NCH, K), jnp.int32),
            pltpu.VMEM((NCH, K), jnp.float32),
        ]
        + [pltpu.VMEM((K, FH), jnp.float32) for _ in range(NB)]
        + [pltpu.VMEM_SHARED((N_PAD, FH), jnp.float32)]
        + [pltpu.SemaphoreType.DMA for _ in range(2 * NB)]
    ),
)(_spmm_body)


def _mm_relu_body(p_ref, w_ref, b_ref, o_ref):
    y = jnp.concatenate([p_ref[0], p_ref[1]], axis=1)
    z = jnp.dot(y, w_ref[...], preferred_element_type=jnp.float32)
    r = jnp.maximum(z + b_ref[...], 0.0)
    o_ref[0] = r[:, :FH]
    o_ref[1] = r[:, FH:]


def _tc_mm_relu(p, w, b):
    rb = 2000
    grid = N_NODES // rb
    return pl.pallas_call(
        _mm_relu_body,
        grid=(grid,),
        in_specs=[
            pl.BlockSpec((NC, rb, FH), lambda i: (0, i, 0)),
            pl.BlockSpec((F, F), lambda i: (0, 0)),
            pl.BlockSpec((1, F), lambda i: (0, 0)),
        ],
        out_specs=pl.BlockSpec((NC, rb, FH), lambda i: (0, i, 0)),
        out_shape=jax.ShapeDtypeStruct((NC, N_NODES, FH), jnp.float32),
    )(p, w, b.reshape(1, F))


def _final_body(p_ref, w2_ref, b2_ref, wh_ref, bh_ref, s_ref, g_ref, acc_ref):
    i = pl.program_id(0)
    y = jnp.concatenate([p_ref[0], p_ref[1]], axis=1)
    z = jnp.dot(y, w2_ref[...], preferred_element_type=jnp.float32)
    h2 = jnp.maximum(z + b2_ref[...], 0.0)
    psum = jnp.sum(h2, axis=0, keepdims=True)

    @pl.when(i == 0)
    def _():
        acc_ref[...] = psum

    @pl.when(i > 0)
    def _():
        acc_ref[...] = acc_ref[...] + psum

    @pl.when(i == pl.num_programs(0) - 1)
    def _():
        g = acc_ref[...] * (1.0 / N_NODES)
        g_ref[...] = g
        s_ref[...] = (
            jnp.dot(g, wh_ref[...], preferred_element_type=jnp.float32)
            + bh_ref[...]
        )


def _tc_final(p, w2, b2, wh, bh):
    rb = 2000
    grid = N_NODES // rb
    nclass = wh.shape[1]
    return pl.pallas_call(
        _final_body,
        grid=(grid,),
        in_specs=[
            pl.BlockSpec((NC, rb, FH), lambda i: (0, i, 0)),
            pl.BlockSpec((F, F), lambda i: (0, 0)),
            pl.BlockSpec((1, F), lambda i: (0, 0)),
            pl.BlockSpec((F, nclass), lambda i: (0, 0)),
            pl.BlockSpec((1, nclass), lambda i: (0, 0)),
        ],
        out_specs=[
            pl.BlockSpec((1, nclass), lambda i: (0, 0)),
            pl.BlockSpec((1, F), lambda i: (0, 0)),
        ],
        out_shape=[
            jax.ShapeDtypeStruct((1, nclass), jnp.float32),
            jax.ShapeDtypeStruct((1, F), jnp.float32),
        ],
        scratch_shapes=[pltpu.VMEM((1, F), jnp.float32)],
    )(p, w2, b2.reshape(1, F), wh, bh.reshape(1, nclass))


def kernel(x, edge_index, edge_vals, W1, b1, W2, b2, Wh, bh):
    dst = edge_index[0].astype(jnp.int32).reshape(NS, NCH, K)
    src = edge_index[1].astype(jnp.int32).reshape(NS, NCH, K)
    ev = edge_vals.reshape(NS, NCH, K)
    zeros = jnp.zeros((N_PAD, FH), jnp.float32)
    p1 = _sc_spmm(x[:, :FH], x[:, FH:], src, dst, ev, zeros)
    h = _tc_mm_relu(p1, W1, b1)
    p2 = _sc_spmm(h[0], h[1], src, dst, ev, zeros)
    scores, g = _tc_final(p2, W2, b2, Wh, bh)
    return (scores, g)


# speedup vs baseline: 1.4687x; 1.4687x over previous
import functools

import jax
import jax.numpy as jnp
from jax import lax
from jax.experimental import pallas as pl
from jax.experimental.pallas import tpu as pltpu
from jax.experimental.pallas import tpu_sc as plsc

N_NODES = 10000
F = 128
FH = F // 2
N_EDGES = 320000
NC = 2
NS = 16
EPW = N_EDGES // NS
K = 80
NCH = EPW // K
N_PAD = 10240
STRIPE = N_PAD // NS


NB = 5
NG = NCH // NB


def _spmm_body(mat0_hbm, mat1_hbm, edge_hbm, vals_hbm,
               out_hbm, src_v, dst_v, vals_v,
               r0, r1, r2, r3, r4, acc_sh,
               g0, g1, g2, g3, g4, s0, s1, s2, s3, s4):
    c = lax.axis_index("c")
    s = lax.axis_index("s")
    rows = [r0, r1, r2, r3, r4]
    gsem = [g0, g1, g2, g3, g4]
    ssem = [s0, s1, s2, s3, s4]

    pltpu.sync_copy(edge_hbm.at[1, s], src_v)
    pltpu.sync_copy(edge_hbm.at[0, s], dst_v)
    pltpu.sync_copy(vals_hbm.at[s], vals_v)

    def zero_row(e, carry):
        for blk in range(FH // 16):
            r0[e, pl.ds(blk * 16, 16)] = jnp.zeros((16,), jnp.float32)
        return carry

    lax.fori_loop(0, K, zero_row, 0)
    for q in range(STRIPE // K):
        pltpu.sync_copy(r0, acc_sh.at[pl.ds(s * STRIPE + q * K, K)])
    plsc.subcore_barrier()

    def gather(j, b):
        @pl.when(c == 0)
        def _():
            pltpu.async_copy(mat0_hbm.at[src_v.at[j]], rows[b], gsem[b])

        @pl.when(c == 1)
        def _():
            pltpu.async_copy(mat1_hbm.at[src_v.at[j]], rows[b], gsem[b])

    def wait_gather(b):
        pltpu.make_async_copy(mat0_hbm.at[src_v.at[0]], rows[b],
                              gsem[b]).wait()

    def scatter(j, b):
        pltpu.async_copy(rows[b], acc_sh.at[dst_v.at[j]], ssem[b],
                         add=True)

    def wait_scatter(b):
        pltpu.make_async_copy(rows[b], acc_sh.at[dst_v.at[0]],
                              ssem[b]).wait()

    def scale(j, b):
        @plsc.parallel_loop(0, K // 16, unroll=K // 16)
        def grp_body(g):
            vv = vals_v[j, pl.ds(g * 16, 16)]
            for l in range(16):
                v = vv[l]
                e = g * 16 + l
                for blk in range(FH // 16):
                    sl = pl.ds(blk * 16, 16)
                    rows[b][e, sl] = rows[b][e, sl] * v

    LEAD = NB - 2

    for b in range(LEAD):
        gather(b, b)

    def group_body(gi, carry):
        base = gi * NB
        for o in range(NB):
            j = base + o
            b = o
            b3 = (o + LEAD) % NB

            wait_gather(b)

            @pl.when(j + LEAD < NCH)
            def _():
                @pl.when(j >= NB - LEAD)
                def _():
                    wait_scatter(b3)

                gather(j + LEAD, b3)

            scale(j, b)
            scatter(j, b)
        return carry

    lax.fori_loop(0, NG, group_body, 0)
    for b in range(NB):
        wait_scatter(b)
    plsc.subcore_barrier()

    pltpu.sync_copy(acc_sh.at[pl.ds(s * STRIPE, STRIPE)],
                    out_hbm.at[c, pl.ds(s * STRIPE, STRIPE)])


_sc_spmm = functools.partial(
    pl.kernel,
    out_type=jax.ShapeDtypeStruct((NC, N_PAD, FH), jnp.float32),
    mesh=plsc.VectorSubcoreMesh(core_axis_name="c", subcore_axis_name="s"),
    compiler_params=pltpu.CompilerParams(use_tc_tiling_on_sc=False),
    scratch_types=(
        [
            pltpu.VMEM((NCH, K), jnp.int32),
            pltpu.VMEM((NCH, K), jnp.int32),
            pltpu.VMEM((NCH, K), jnp.float32),
        ]
        + [pltpu.VMEM((K, FH), jnp.float32) for _ in range(NB)]
        + [pltpu.VMEM_SHARED((N_PAD, FH), jnp.float32)]
        + [pltpu.SemaphoreType.DMA for _ in range(2 * NB)]
    ),
)(_spmm_body)


def _mm_relu_body(p_ref, w_ref, b_ref, o_ref):
    y = jnp.concatenate([p_ref[0], p_ref[1]], axis=1)
    z = jnp.dot(y, w_ref[...], preferred_element_type=jnp.float32)
    r = jnp.maximum(z + b_ref[...], 0.0)
    o_ref[0] = r[:, :FH]
    o_ref[1] = r[:, FH:]


def _tc_mm_relu(p, w, b):
    rb = 2000
    grid = N_NODES // rb
    return pl.pallas_call(
        _mm_relu_body,
        grid=(grid,),
        in_specs=[
            pl.BlockSpec((NC, rb, FH), lambda i: (0, i, 0)),
            pl.BlockSpec((F, F), lambda i: (0, 0)),
            pl.BlockSpec((1, F), lambda i: (0, 0)),
        ],
        out_specs=pl.BlockSpec((NC, rb, FH), lambda i: (0, i, 0)),
        out_shape=jax.ShapeDtypeStruct((NC, N_NODES, FH), jnp.float32),
    )(p, w, b.reshape(1, F))


def _final_body(p_ref, w2_ref, b2_ref, wh_ref, bh_ref, s_ref, g_ref, acc_ref):
    i = pl.program_id(0)
    y = jnp.concatenate([p_ref[0], p_ref[1]], axis=1)
    z = jnp.dot(y, w2_ref[...], preferred_element_type=jnp.float32)
    h2 = jnp.maximum(z + b2_ref[...], 0.0)
    psum = jnp.sum(h2, axis=0, keepdims=True)

    @pl.when(i == 0)
    def _():
        acc_ref[...] = psum

    @pl.when(i > 0)
    def _():
        acc_ref[...] = acc_ref[...] + psum

    @pl.when(i == pl.num_programs(0) - 1)
    def _():
        g = acc_ref[...] * (1.0 / N_NODES)
        g_ref[...] = g
        s_ref[...] = (
            jnp.dot(g, wh_ref[...], preferred_element_type=jnp.float32)
            + bh_ref[...]
        )


def _tc_final(p, w2, b2, wh, bh):
    rb = 2000
    grid = N_NODES // rb
    nclass = wh.shape[1]
    return pl.pallas_call(
        _final_body,
        grid=(grid,),
        in_specs=[
            pl.BlockSpec((NC, rb, FH), lambda i: (0, i, 0)),
            pl.BlockSpec((F, F), lambda i: (0, 0)),
            pl.BlockSpec((1, F), lambda i: (0, 0)),
            pl.BlockSpec((F, nclass), lambda i: (0, 0)),
            pl.BlockSpec((1, nclass), lambda i: (0, 0)),
        ],
        out_specs=[
            pl.BlockSpec((1, nclass), lambda i: (0, 0)),
            pl.BlockSpec((1, F), lambda i: (0, 0)),
        ],
        out_shape=[
            jax.ShapeDtypeStruct((1, nclass), jnp.float32),
            jax.ShapeDtypeStruct((1, F), jnp.float32),
        ],
        scratch_shapes=[pltpu.VMEM((1, F), jnp.float32)],
    )(p, w2, b2.reshape(1, F), wh, bh.reshape(1, nclass))


def kernel(x, edge_index, edge_vals, W1, b1, W2, b2, Wh, bh):
    edges = edge_index.astype(jnp.int32).reshape(2, NS, NCH, K)
    ev = edge_vals.reshape(NS, NCH, K)
    p1 = _sc_spmm(x[:, :FH], x[:, FH:], edges, ev)
    h = _tc_mm_relu(p1, W1, b1)
    p2 = _sc_spmm(h[0], h[1], edges, ev)
    scores, g = _tc_final(p2, W2, b2, Wh, bh)
    return (scores, g)


# speedup vs baseline: 1.4895x; 1.0142x over previous
import functools

import jax
import jax.numpy as jnp
from jax import lax
from jax.experimental import pallas as pl
from jax.experimental.pallas import tpu as pltpu
from jax.experimental.pallas import tpu_sc as plsc

N_NODES = 10000
F = 128
FH = F // 2
N_EDGES = 320000
NC = 2
NS = 16
EPW = N_EDGES // NS
K = 80
NCH = EPW // K
N_PAD = 10240
STRIPE = N_PAD // NS


NB = 5
NG = NCH // NB


def _spmm_body(mat0_hbm, mat1_hbm, edge_hbm, vals_hbm,
               out_hbm, src_v, dst_v, vals_v,
               r0, r1, r2, r3, r4, acc_sh,
               g0, g1, g2, g3, g4, s0, s1, s2, s3, s4):
    c = lax.axis_index("c")
    s = lax.axis_index("s")
    rows = [r0, r1, r2, r3, r4]
    gsem = [g0, g1, g2, g3, g4]
    ssem = [s0, s1, s2, s3, s4]

    pltpu.sync_copy(edge_hbm.at[1, s], src_v)
    pltpu.sync_copy(edge_hbm.at[0, s], dst_v)
    pltpu.sync_copy(vals_hbm.at[s], vals_v)

    def zero_row(e, carry):
        for blk in range(FH // 16):
            r0[e, pl.ds(blk * 16, 16)] = jnp.zeros((16,), jnp.float32)
        return carry

    lax.fori_loop(0, K, zero_row, 0)
    for q in range(STRIPE // K):
        pltpu.sync_copy(r0, acc_sh.at[pl.ds(s * STRIPE + q * K, K)])
    plsc.subcore_barrier()

    def gather(j, b):
        @pl.when(c == 0)
        def _():
            pltpu.async_copy(mat0_hbm.at[src_v.at[j]], rows[b], gsem[b])

        @pl.when(c == 1)
        def _():
            pltpu.async_copy(mat1_hbm.at[src_v.at[j]], rows[b], gsem[b])

    def wait_gather(b):
        pltpu.make_async_copy(mat0_hbm.at[src_v.at[0]], rows[b],
                              gsem[b]).wait()

    def scatter(j, b):
        pltpu.async_copy(rows[b], acc_sh.at[dst_v.at[j]], ssem[b],
                         add=True)

    def wait_scatter(b):
        pltpu.make_async_copy(rows[b], acc_sh.at[dst_v.at[0]],
                              ssem[b]).wait()

    def scale(j, b):
        @plsc.parallel_loop(0, K // 16, unroll=K // 16)
        def grp_body(g):
            vv = vals_v[j, pl.ds(g * 16, 16)]
            for l in range(16):
                v = vv[l]
                e = g * 16 + l
                for blk in range(FH // 16):
                    sl = pl.ds(blk * 16, 16)
                    rows[b][e, sl] = rows[b][e, sl] * v

    LEAD = NB - 2

    for b in range(LEAD):
        gather(b, b)

    def group_body(gi, carry):
        base = gi * NB
        for o in range(NB):
            j = base + o
            b = o
            b3 = (o + LEAD) % NB

            wait_gather(b)

            @pl.when(j + LEAD < NCH)
            def _():
                @pl.when(j >= NB - LEAD)
                def _():
                    wait_scatter(b3)

                gather(j + LEAD, b3)

            scale(j, b)
            scatter(j, b)
        return carry

    lax.fori_loop(0, NG, group_body, 0)
    for b in range(NB):
        wait_scatter(b)
    plsc.subcore_barrier()

    pltpu.sync_copy(acc_sh.at[pl.ds(s * STRIPE, STRIPE)],
                    out_hbm.at[c, pl.ds(s * STRIPE, STRIPE)])


_sc_spmm = functools.partial(
    pl.kernel,
    out_type=jax.ShapeDtypeStruct((NC, N_PAD, FH), jnp.float32),
    mesh=plsc.VectorSubcoreMesh(core_axis_name="c", subcore_axis_name="s"),
    compiler_params=pltpu.CompilerParams(use_tc_tiling_on_sc=False),
    scratch_types=(
        [
            pltpu.VMEM((NCH, K), jnp.int32),
            pltpu.VMEM((NCH, K), jnp.int32),
            pltpu.VMEM((NCH, K), jnp.float32),
        ]
        + [pltpu.VMEM((K, FH), jnp.float32) for _ in range(NB)]
        + [pltpu.VMEM_SHARED((N_PAD, FH), jnp.float32)]
        + [pltpu.SemaphoreType.DMA for _ in range(2 * NB)]
    ),
)(_spmm_body)


def _mm_relu_body(p_ref, w_ref, b_ref, o0_ref, o1_ref):
    y = jnp.concatenate([p_ref[0], p_ref[1]], axis=1)
    z = jnp.dot(y, w_ref[...], preferred_element_type=jnp.float32)
    r = jnp.maximum(z + b_ref[...], 0.0)
    o0_ref[...] = r[:, :FH]
    o1_ref[...] = r[:, FH:]


def _tc_mm_relu(p, w, b):
    rb = 2000
    grid = N_NODES // rb
    return pl.pallas_call(
        _mm_relu_body,
        grid=(grid,),
        in_specs=[
            pl.BlockSpec((NC, rb, FH), lambda i: (0, i, 0)),
            pl.BlockSpec((F, F), lambda i: (0, 0)),
            pl.BlockSpec((1, F), lambda i: (0, 0)),
        ],
        out_specs=[
            pl.BlockSpec((rb, FH), lambda i: (i, 0)),
            pl.BlockSpec((rb, FH), lambda i: (i, 0)),
        ],
        out_shape=[
            jax.ShapeDtypeStruct((N_NODES, FH), jnp.float32),
            jax.ShapeDtypeStruct((N_NODES, FH), jnp.float32),
        ],
    )(p, w, b.reshape(1, F))


def _final_body(p_ref, w2_ref, b2_ref, wh_ref, bh_ref, s_ref, g_ref, acc_ref):
    i = pl.program_id(0)
    y = jnp.concatenate([p_ref[0], p_ref[1]], axis=1)
    z = jnp.dot(y, w2_ref[...], preferred_element_type=jnp.float32)
    h2 = jnp.maximum(z + b2_ref[...], 0.0)
    psum = jnp.sum(h2, axis=0, keepdims=True)

    @pl.when(i == 0)
    def _():
        acc_ref[...] = psum

    @pl.when(i > 0)
    def _():
        acc_ref[...] = acc_ref[...] + psum

    @pl.when(i == pl.num_programs(0) - 1)
    def _():
        g = acc_ref[...] * (1.0 / N_NODES)
        g_ref[...] = g
        s_ref[...] = (
            jnp.dot(g, wh_ref[...], preferred_element_type=jnp.float32)
            + bh_ref[...]
        )


def _tc_final(p, w2, b2, wh, bh):
    rb = 2000
    grid = N_NODES // rb
    nclass = wh.shape[1]
    return pl.pallas_call(
        _final_body,
        grid=(grid,),
        in_specs=[
            pl.BlockSpec((NC, rb, FH), lambda i: (0, i, 0)),
            pl.BlockSpec((F, F), lambda i: (0, 0)),
            pl.BlockSpec((1, F), lambda i: (0, 0)),
            pl.BlockSpec((F, nclass), lambda i: (0, 0)),
            pl.BlockSpec((1, nclass), lambda i: (0, 0)),
        ],
        out_specs=[
            pl.BlockSpec((1, nclass), lambda i: (0, 0)),
            pl.BlockSpec((1, F), lambda i: (0, 0)),
        ],
        out_shape=[
            jax.ShapeDtypeStruct((1, nclass), jnp.float32),
            jax.ShapeDtypeStruct((1, F), jnp.float32),
        ],
        scratch_shapes=[pltpu.VMEM((1, F), jnp.float32)],
    )(p, w2, b2.reshape(1, F), wh, bh.reshape(1, nclass))


def kernel(x, edge_index, edge_vals, W1, b1, W2, b2, Wh, bh):
    edges = edge_index.astype(jnp.int32).reshape(2, NS, NCH, K)
    ev = edge_vals.reshape(NS, NCH, K)
    p1 = _sc_spmm(x[:, :FH], x[:, FH:], edges, ev)
    h0, h1 = _tc_mm_relu(p1, W1, b1)
    p2 = _sc_spmm(h0, h1, edges, ev)
    scores, g = _tc_final(p2, W2, b2, Wh, bh)
    return (scores, g)
